# baseline (speedup 1.0000x reference)
import functools

import jax
import jax.numpy as jnp
from jax.experimental import pallas as pl

_DIM = 64
_HEADS = 8
_DH = 64
_INNER = _HEADS * _DH
_N = 128
_CBS = 4
_NC = _N // _CBS
_NSEL = 2
_WIN = 2
_FF = 256
_BATCH = 512

_NEG = -1e9
_SCALE = _DH ** -0.5


def _block_mean4(a):
    return a.reshape(_NC, _CBS, _DH).mean(axis=1)


def _ln(x, g, b, eps=1e-5):
    m = x.mean(-1, keepdims=True)
    v = ((x - m) ** 2).mean(-1, keepdims=True)
    return (x - m) / jnp.sqrt(v + eps) * g + b


def _tabnsa_kernel(
    xt_ref,
    w_emb_ref, b_emb_ref,
    wq_ref, wk_ref, wv_ref,
    wg_ref, bg_ref,
    wo_ref,
    ln1g_ref, ln1b_ref, ln2g_ref, ln2b_ref,
    wt1t_ref, bt1c_ref, wt2t_ref, bt2c_ref,
    wc1_ref, bc1_ref, wc2_ref, bc2_ref,
    wh1_ref, bh1_ref, wh2_ref, bh2_ref,
    o_ref,
):
    f32 = jnp.float32
    row = jax.lax.broadcasted_iota(jnp.int32, (_N, _N), 0)
    col = jax.lax.broadcasted_iota(jnp.int32, (_N, _N), 1)
    causal = col <= row
    winm = causal & ((row - col) < _WIN)
    blk = col // _CBS
    jj = jax.lax.broadcasted_iota(jnp.int32, (_N, _NC), 1)
    ii = jax.lax.broadcasted_iota(jnp.int32, (_N, _NC), 0)
    cmask = (jj * _CBS + (_CBS - 1)) <= ii

    xcol = xt_ref[...].reshape(_N, 1)
    e = xcol * w_emb_ref[...] + b_emb_ref[...]

    q_all = jnp.dot(e, wq_ref[...])
    k_all = jnp.dot(e, wk_ref[...])
    v_all = jnp.dot(e, wv_ref[...])
    gates = jax.nn.sigmoid(jnp.dot(e, wg_ref[...]) + bg_ref[...])

    attn_heads = []
    for h in range(_HEADS):
        s0 = h * _DH
        q = q_all[:, s0:s0 + _DH]
        k = k_all[:, s0:s0 + _DH]
        v = v_all[:, s0:s0 + _DH]
        kc = _block_mean4(k)
        vc = _block_mean4(v)
        k_ext = jnp.concatenate([k, kc], axis=0)
        s_ext = jax.lax.dot_general(
            q, k_ext, (((1,), (1,)), ((), ()))) * _SCALE
        s_full = s_ext[:, :_N]
        sc = s_ext[:, _N:]

        sc_m = jnp.where(cmask, sc, _NEG)
        mc = jnp.max(sc_m, axis=1, keepdims=True)
        ec = jnp.exp(sc_m - mc)
        pc = ec / jnp.sum(ec, axis=1, keepdims=True)

        m1 = jnp.max(sc_m, axis=1, keepdims=True)
        idx1 = jnp.min(jnp.where(sc_m == m1, jj, _NC), axis=1, keepdims=True)
        sc_m2 = jnp.where(jj == idx1, jnp.finfo(f32).min, sc_m)
        m2 = jnp.max(sc_m2, axis=1, keepdims=True)
        idx2 = jnp.min(jnp.where(sc_m2 == m2, jj, _NC), axis=1, keepdims=True)

        s_c = jnp.where(causal, s_full, _NEG)
        mrow = jnp.max(s_c, axis=1, keepdims=True)
        e_c = jnp.where(causal, jnp.exp(s_c - mrow), 0.0)

        fsel = (blk == idx1) | (blk == idx2)
        w_f = jnp.where(fsel, e_c, 0.0)
        d_f = jnp.sum(w_f, axis=1, keepdims=True)

        w_w = jnp.where(winm, e_c, 0.0)
        d_w = jnp.sum(w_w, axis=1, keepdims=True)

        g0 = gates[:, h:h + 1]
        g1 = gates[:, _HEADS + h:_HEADS + h + 1]
        g2 = gates[:, 2 * _HEADS + h:2 * _HEADS + h + 1]
        p_fw = (g1 / d_f) * w_f + (g2 / d_w) * w_w
        p_c = g0 * pc
        p_all = jnp.concatenate([p_fw, p_c], axis=1)
        v_ext = jnp.concatenate([v, vc], axis=0)
        attn_heads.append(jnp.dot(p_all, v_ext))

    attn = jnp.concatenate(attn_heads, axis=1)
    attn_out = jnp.dot(attn, wo_ref[...])

    t = _ln(e, ln1g_ref[...], ln1b_ref[...])
    a1 = jax.nn.gelu(jnp.dot(wt1t_ref[...], t) + bt1c_ref[...])
    tmix = jnp.dot(wt2t_ref[...], a1) + bt2c_ref[...]
    y = e + tmix
    c_in = _ln(y, ln2g_ref[...], ln2b_ref[...])
    c1 = jax.nn.gelu(jnp.dot(c_in, wc1_ref[...]) + bc1_ref[...])
    cmix = jnp.dot(c1, wc2_ref[...]) + bc2_ref[...]
    mix = y + cmix

    pooled = jnp.mean(attn_out + mix, axis=0, keepdims=True)
    h1 = jax.nn.gelu(jnp.dot(pooled, wh1_ref[...]) + bh1_ref[...])
    out = jnp.dot(h1, wh2_ref[...]) + bh2_ref[...]
    o_ref[...] = out.reshape(1, 1, 1)


@jax.jit
def kernel(x, params):
    p = params
    xt = x.reshape(_BATCH, _N, 1)
    row2 = lambda a: a.reshape(1, -1)
    col2 = lambda a: a.reshape(-1, 1)
    ins = (
        xt,
        p['W_emb'], row2(p['b_emb']),
        p['Wq'], p['Wk'], p['Wv'],
        p['Wg'], row2(p['bg']),
        p['Wo'],
        row2(p['ln1_g']), row2(p['ln1_b']), row2(p['ln2_g']), row2(p['ln2_b']),
        p['Wt1'].T, col2(p['bt1']), p['Wt2'].T, col2(p['bt2']),
        p['Wc1'], row2(p['bc1']), p['Wc2'], row2(p['bc2']),
        p['Wh1'], row2(p['bh1']), p['Wh2'], row2(p['bh2']),
    )

    def const_spec(a):
        nd = a.ndim
        return pl.BlockSpec(a.shape, lambda i, _nd=nd: (0,) * _nd)

    in_specs = [pl.BlockSpec((1, _N, 1), lambda i: (i, 0, 0))]
    in_specs += [const_spec(a) for a in ins[1:]]

    out = pl.pallas_call(
        _tabnsa_kernel,
        grid=(_BATCH,),
        in_specs=in_specs,
        out_specs=pl.BlockSpec((1, 1, 1), lambda i: (i, 0, 0)),
        out_shape=jax.ShapeDtypeStruct((_BATCH, 1, 1), jnp.float32),
    )(*ins)
    return out.reshape(_BATCH, 1)
